# baseline (speedup 1.0000x reference)
import functools

import jax
import jax.numpy as jnp
from jax import lax
from jax.experimental import pallas as pl
from jax.experimental.pallas import tpu as pltpu
from jax.experimental.pallas import tpu_sc as plsc

VOCAB = 100000
D = 64
B = 4096
S = 200

NC = 2
NS = 16
NW = NC * NS
BPW = B // NW
NCHUNK = S
NREG = D // 16
NBUF = 8
RED_UNROLL = 4


def _bow_body(inputs_hbm, table_hbm, bias_hbm, out_hbm,
              idx_v, idx_t, rows_v, acc_v, bias_v, *sems):
    wid = lax.axis_index("s") * NC + lax.axis_index("c")
    base = wid * BPW

    pltpu.sync_copy(inputs_hbm.at[pl.ds(base, BPW)], idx_v)
    pltpu.sync_copy(bias_hbm, bias_v)
    bias_regs = [bias_v[pl.ds(16 * d, 16)] for d in range(NREG)]

    def init_body(t, carry):
        for u in range(RED_UNROLL):
            for d in range(NREG):
                acc_v[RED_UNROLL * t + u, pl.ds(16 * d, 16)] = bias_regs[d]
        return carry
    lax.fori_loop(0, BPW // RED_UNROLL, init_body, 0)

    row_ids = [lax.iota(jnp.int32, 16) + 16 * k for k in range(BPW // 16)]

    def build_col(c):
        col = jnp.full((16,), c, jnp.int32)
        for k in range(BPW // 16):
            idx_t[c, pl.ds(16 * k, 16)] = plsc.load_gather(idx_v, [row_ids[k], col])

    def issue(c, slot):
        pltpu.async_copy(table_hbm.at[idx_t.at[c]], rows_v.at[slot], sems[slot])

    for c in range(NBUF - 1):
        build_col(c)
        issue(c, c)

    def build_body(c, carry):
        build_col(c)
        return carry
    lax.fori_loop(NBUF - 1, NCHUNK, build_body, 0)

    def group_body(g, carry):
        for b in range(NBUF):
            c = g * NBUF + b
            c_next = c + NBUF - 1
            slot_next = (b + NBUF - 1) % NBUF

            @pl.when(c_next < NCHUNK)
            def _():
                issue(c_next, slot_next)

            pltpu.make_async_copy(table_hbm.at[idx_t.at[c]],
                                  rows_v.at[b], sems[b]).wait()

            def red_body(t, carry2):
                for u in range(RED_UNROLL):
                    tt = RED_UNROLL * t + u
                    for d in range(NREG):
                        plsc.addupdate(acc_v.at[tt, pl.ds(16 * d, 16)],
                                       rows_v[b, tt, pl.ds(16 * d, 16)])
                return carry2
            lax.fori_loop(0, BPW // RED_UNROLL, red_body, 0)
        return carry

    lax.fori_loop(0, NCHUNK // NBUF, group_body, 0)
    pltpu.sync_copy(acc_v, out_hbm.at[pl.ds(base, BPW)])


def _bow(inputs, table, bias):
    mesh = plsc.VectorSubcoreMesh(core_axis_name="c", subcore_axis_name="s")
    kern = functools.partial(
        pl.kernel,
        mesh=mesh,
        out_type=jax.ShapeDtypeStruct((B, D), jnp.float32),
        scratch_types=[
            pltpu.VMEM((BPW, S), jnp.int32),
            pltpu.VMEM((S, BPW), jnp.int32),
            pltpu.VMEM((NBUF, BPW, D), jnp.float32),
            pltpu.VMEM((BPW, D), jnp.float32),
            pltpu.VMEM((D,), jnp.float32),
        ] + [pltpu.SemaphoreType.DMA] * NBUF,
        compiler_params=pltpu.CompilerParams(use_tc_tiling_on_sc=False,
                                             needs_layout_passes=False),
    )(_bow_body)
    return kern(inputs, table, bias)


def kernel(inputs, embed_weight, bias):
    return _bow(inputs.astype(jnp.int32), embed_weight, bias)


# speedup vs baseline: 1.1589x; 1.1589x over previous
import functools

import jax
import jax.numpy as jnp
from jax import lax
from jax.experimental import pallas as pl
from jax.experimental.pallas import tpu as pltpu
from jax.experimental.pallas import tpu_sc as plsc

VOCAB = 100000
D = 64
B = 4096
S = 200

NC = 2
NS = 16
NW = NC * NS
BPW = B // NW
NCHUNK = S
NREG = D // 16
NBUF = 8
RED_UNROLL = 4


def _bow_body(inputs_hbm, table_hbm, bias_hbm, out_hbm,
              idx_v, idx_t, rows_v, acc_v, bias_v, *sems):
    wid = lax.axis_index("s") * NC + lax.axis_index("c")
    base = wid * BPW

    pltpu.sync_copy(inputs_hbm.at[pl.ds(base, BPW)], idx_v)
    pltpu.sync_copy(bias_hbm, bias_v)
    bias_regs = [bias_v[pl.ds(16 * d, 16)] for d in range(NREG)]

    def init_body(t, carry):
        for u in range(RED_UNROLL):
            for d in range(NREG):
                acc_v[RED_UNROLL * t + u, pl.ds(16 * d, 16)] = bias_regs[d]
        return carry
    lax.fori_loop(0, BPW // RED_UNROLL, init_body, 0)

    row_ids = [lax.iota(jnp.int32, 16) + 16 * k for k in range(BPW // 16)]

    def build_col(c):
        col = jnp.full((16,), c, jnp.int32)
        for k in range(BPW // 16):
            idx_t[c, pl.ds(16 * k, 16)] = plsc.load_gather(idx_v, [row_ids[k], col])

    def issue(c, slot):
        pltpu.async_copy(table_hbm.at[idx_t.at[c]], rows_v.at[slot], sems[slot])

    for c in range(NBUF):
        build_col(c)
        issue(c, c)

    def build_body(c, carry):
        build_col(c)
        return carry
    lax.fori_loop(NBUF, NCHUNK, build_body, 0)

    QUAD = 4

    def quad(g, q):
        c0 = g * NBUF + q * QUAD
        slots = range(q * QUAD, (q + 1) * QUAD)
        for k, s in enumerate(slots):
            pltpu.make_async_copy(table_hbm.at[idx_t.at[c0 + k]],
                                  rows_v.at[s], sems[s]).wait()

        def red_body(t, carry2):
            for u in range(RED_UNROLL):
                tt = RED_UNROLL * t + u
                for d in range(NREG):
                    sl = pl.ds(16 * d, 16)
                    s0, s1, s2, s3 = slots
                    v = ((rows_v[s0, tt, sl] + rows_v[s1, tt, sl])
                         + (rows_v[s2, tt, sl] + rows_v[s3, tt, sl]))
                    plsc.addupdate(acc_v.at[tt, sl], v)
            return carry2
        lax.fori_loop(0, BPW // RED_UNROLL, red_body, 0)

        @pl.when(c0 + NBUF < NCHUNK)
        def _():
            for k, s in enumerate(slots):
                issue(c0 + NBUF + k, s)

    def group_body(g, carry):
        quad(g, 0)
        quad(g, 1)
        return carry

    lax.fori_loop(0, NCHUNK // NBUF, group_body, 0)
    pltpu.sync_copy(acc_v, out_hbm.at[pl.ds(base, BPW)])


def _bow(inputs, table, bias):
    mesh = plsc.VectorSubcoreMesh(core_axis_name="c", subcore_axis_name="s")
    kern = functools.partial(
        pl.kernel,
        mesh=mesh,
        out_type=jax.ShapeDtypeStruct((B, D), jnp.float32),
        scratch_types=[
            pltpu.VMEM((BPW, S), jnp.int32),
            pltpu.VMEM((S, BPW), jnp.int32),
            pltpu.VMEM((NBUF, BPW, D), jnp.float32),
            pltpu.VMEM((BPW, D), jnp.float32),
            pltpu.VMEM((D,), jnp.float32),
        ] + [pltpu.SemaphoreType.DMA] * NBUF,
        compiler_params=pltpu.CompilerParams(use_tc_tiling_on_sc=False,
                                             needs_layout_passes=False),
    )(_bow_body)
    return kern(inputs, table, bias)


def kernel(inputs, embed_weight, bias):
    return _bow(inputs.astype(jnp.int32), embed_weight, bias)


# speedup vs baseline: 1.2474x; 1.0763x over previous
import functools

import jax
import jax.numpy as jnp
from jax import lax
from jax.experimental import pallas as pl
from jax.experimental.pallas import tpu as pltpu
from jax.experimental.pallas import tpu_sc as plsc

VOCAB = 100000
D = 64
B = 4096
S = 200

NC = 2
NS = 16
NW = NC * NS
B_PER_W = B // NW
HALF = S // 2
NREG = D // 16


NBUF = 4


def _bow_body(inputs_hbm, table_hbm, bias_hbm, out_hbm,
              idx_v, rows_v, out_v, bias_v, *sems):
    wid = lax.axis_index("s") * NC + lax.axis_index("c")
    base = wid * B_PER_W

    pltpu.sync_copy(inputs_hbm.at[pl.ds(base, B_PER_W)], idx_v)
    pltpu.sync_copy(bias_hbm, bias_v)
    bias_regs = [bias_v[pl.ds(16 * d, 16)] for d in range(NREG)]

    def issue(r, slot):
        for j in range(2):
            pltpu.async_copy(table_hbm.at[idx_v.at[r, j]],
                             rows_v.at[slot, j], sems[slot])

    for r in range(NBUF - 1):
        issue(r, r)

    def group_body(g, carry):
        for b in range(NBUF):
            r = g * NBUF + b
            r_next = r + NBUF - 1
            slot_next = (b + NBUF - 1) % NBUF

            @pl.when(r_next < B_PER_W)
            def _():
                issue(r_next, slot_next)

            for j in range(2):
                pltpu.make_async_copy(table_hbm.at[idx_v.at[r, j]],
                                      rows_v.at[b, j], sems[b]).wait()

            def seq_body(t, acc):
                new = list(acc)
                for j in range(2):
                    for d in range(NREG):
                        new[j * NREG + d] = (new[j * NREG + d]
                                             + rows_v[b, j, t, pl.ds(16 * d, 16)])
                return tuple(new)

            zero = jnp.zeros((16,), jnp.float32)
            acc = lax.fori_loop(0, HALF, seq_body, (zero,) * (2 * NREG))
            for d in range(NREG):
                out_v[r, pl.ds(16 * d, 16)] = acc[d] + acc[NREG + d] + bias_regs[d]
        return carry

    lax.fori_loop(0, B_PER_W // NBUF, group_body, 0)
    pltpu.sync_copy(out_v, out_hbm.at[pl.ds(base, B_PER_W)])


def _bow(inputs3, table, bias):
    mesh = plsc.VectorSubcoreMesh(core_axis_name="c", subcore_axis_name="s")
    kern = functools.partial(
        pl.kernel,
        mesh=mesh,
        out_type=jax.ShapeDtypeStruct((B, D), jnp.float32),
        scratch_types=[
            pltpu.VMEM((B_PER_W, 2, HALF), jnp.int32),
            pltpu.VMEM((NBUF, 2, HALF, D), jnp.float32),
            pltpu.VMEM((B_PER_W, D), jnp.float32),
            pltpu.VMEM((D,), jnp.float32),
        ] + [pltpu.SemaphoreType.DMA] * NBUF,
        compiler_params=pltpu.CompilerParams(use_tc_tiling_on_sc=False),
    )(_bow_body)
    return kern(inputs3, table, bias)


def kernel(inputs, embed_weight, bias):
    inputs3 = inputs.astype(jnp.int32).reshape(B, 2, HALF)
    return _bow(inputs3, embed_weight, bias)
